# baseline (speedup 1.0000x reference)
import functools

import jax
import jax.numpy as jnp
from jax import lax
from jax.experimental import pallas as pl
from jax.experimental.pallas import tpu as pltpu
from jax.experimental.pallas import tpu_sc as plsc

B, S, D = 8, 4096, 1024
NSC = 4
NTC = B - NSC
NC, NS = 2, 16
NW = NC * NS
CH = 32
NCHUNK = S // CH // NW
ZH = 16
PH = 16
SSTRIP = 512
NSTRIP = S // SSTRIP
L = 16
BS = 512


def _sc_body(seg_hbm, table_hbm, out_hbm,
             t0_v, t1_v, z_v, p0_v, p1_v, sg0_v, sg1_v,
             idx0_v, idx1_v, pidx0_v, pidx1_v,
             sem_l, sem_s, sem_w, sem_p, sem_b):
    wid = lax.axis_index("s") * NC + lax.axis_index("c")
    tbufs = [t0_v, t1_v]
    idxbufs = [idx0_v, idx1_v]
    sgbufs = [sg0_v, sg1_v]
    lane = lax.iota(jnp.int32, L)

    def chunk_j0(q):
        return (q * NW + wid) * CH

    def build_idx(ref, j0):
        ref[pl.ds(0, L)] = lane + (j0 + 2)
        ref[pl.ds(L, L)] = lane + (j0 + 2 + L)

    build_idx(idx0_v, chunk_j0(0))
    pltpu.async_copy(table_hbm.at[idx0_v], t0_v, sem_l)
    pltpu.async_copy(seg_hbm.at[:, pl.ds(0, SSTRIP)], sg0_v, sem_s)

    zvec = jnp.zeros((L,), jnp.float32)

    def zrow(r, _):
        def zcol(k, _):
            z_v[r, pl.ds(k * 4 * L, L)] = zvec
            z_v[r, pl.ds((k * 4 + 1) * L, L)] = zvec
            z_v[r, pl.ds((k * 4 + 2) * L, L)] = zvec
            z_v[r, pl.ds((k * 4 + 3) * L, L)] = zvec
            return 0
        return lax.fori_loop(0, D // L // 4, zcol, 0)

    lax.fori_loop(0, ZH, zrow, 0)

    accs = tuple(jnp.zeros((L,), jnp.int32) for _ in range(NSC))
    for k in range(NSTRIP):
        sg = sgbufs[k % 2]
        pltpu.make_async_copy(
            seg_hbm.at[:, pl.ds(k * SSTRIP, SSTRIP)], sg, sem_s).wait()
        if k + 1 < NSTRIP:
            pltpu.async_copy(
                seg_hbm.at[:, pl.ds((k + 1) * SSTRIP, SSTRIP)],
                sgbufs[(k + 1) % 2], sem_s)

        def red(t, acc):
            return tuple(acc[i] + sg[NTC + i, pl.ds(t * L, L)]
                         for i in range(NSC))

        accs = lax.fori_loop(0, SSTRIP // L, red, accs)
    counts = [jnp.sum(a) for a in accs]

    def drain_bnd(n):
        def w(_, carry):
            pltpu.make_async_copy(p0_v, out_hbm.at[0, pl.ds(0, PH)],
                                  sem_b).wait()
            return carry
        lax.fori_loop(0, n, w, 0)

    bfired = jnp.int32(0)
    bdrained = jnp.int32(0)
    for i in range(NSC):
        c = counts[i]
        g = c // CH
        mine = jnp.logical_and(c % CH != 0, g % NW == wid)
        drain_bnd(bfired - bdrained)
        bdrained = bfired

        @pl.when(mine)
        def _():
            j0b = g * CH
            jv0 = lane + j0b
            pidx0_v[pl.ds(0, L)] = jnp.where(jv0 < c, jv0 + 2, 0)
            jv1 = lane + (j0b + PH)
            pidx1_v[pl.ds(0, L)] = jnp.where(jv1 < c, jv1 + 2, 0)
            cp0 = pltpu.async_copy(table_hbm.at[pidx0_v], p0_v, sem_p)
            cp1 = pltpu.async_copy(table_hbm.at[pidx1_v], p1_v, sem_p)
            cp0.wait()
            cp1.wait()
            pltpu.async_copy(p0_v, out_hbm.at[i, pl.ds(j0b, PH)], sem_b)
            pltpu.async_copy(p1_v, out_hbm.at[i, pl.ds(j0b + PH, PH)],
                             sem_b)

        bfired = bfired + jnp.where(mine, 2, 0).astype(jnp.int32)

    def drain_writes(n):
        def w(_, carry):
            pltpu.make_async_copy(z_v, out_hbm.at[0, pl.ds(0, ZH)],
                                  sem_w).wait()
            return carry
        lax.fori_loop(0, n, w, 0)

    fired = jnp.int32(0)
    drained = jnp.int32(0)

    for q in range(NCHUNK):
        t_v = tbufs[q % 2]
        j0 = chunk_j0(q)
        pltpu.make_async_copy(table_hbm.at[idxbufs[q % 2]], t_v,
                              sem_l).wait()
        if q + 1 < NCHUNK:
            drain_writes(fired - drained)
            drained = fired
            build_idx(idxbufs[(q + 1) % 2], chunk_j0(q + 1))
            pltpu.async_copy(table_hbm.at[idxbufs[(q + 1) % 2]],
                             tbufs[(q + 1) % 2], sem_l)

        for i in range(NSC):
            c = counts[i]
            full = (j0 + CH) <= c
            empty = j0 >= c
            boundary = jnp.logical_and(jnp.logical_not(full),
                                       jnp.logical_not(empty))

            @pl.when(full)
            def _():
                pltpu.async_copy(t_v, out_hbm.at[i, pl.ds(j0, CH)], sem_w)

            @pl.when(empty)
            def _():
                pltpu.async_copy(z_v, out_hbm.at[i, pl.ds(j0, ZH)], sem_w)
                pltpu.async_copy(z_v, out_hbm.at[i, pl.ds(j0 + ZH, ZH)],
                                 sem_w)

            fired = fired + jnp.where(boundary, 0, 2).astype(jnp.int32)

    drain_writes(fired - drained)
    drain_bnd(bfired - bdrained)


def _sc_run(seg, emb_table):
    mesh = plsc.VectorSubcoreMesh(core_axis_name="c", subcore_axis_name="s")
    f = pl.kernel(
        _sc_body,
        out_type=jax.ShapeDtypeStruct((NSC, S, D), jnp.float32),
        mesh=mesh,
        scratch_types=[
            pltpu.VMEM((CH, D), jnp.float32),
            pltpu.VMEM((CH, D), jnp.float32),
            pltpu.VMEM((ZH, D), jnp.float32),
            pltpu.VMEM((PH, D), jnp.float32),
            pltpu.VMEM((PH, D), jnp.float32),
            pltpu.VMEM((B, SSTRIP), jnp.int32),
            pltpu.VMEM((B, SSTRIP), jnp.int32),
            pltpu.VMEM((CH,), jnp.int32),
            pltpu.VMEM((CH,), jnp.int32),
            pltpu.VMEM((PH,), jnp.int32),
            pltpu.VMEM((PH,), jnp.int32),
            pltpu.SemaphoreType.DMA,
            pltpu.SemaphoreType.DMA,
            pltpu.SemaphoreType.DMA,
            pltpu.SemaphoreType.DMA,
            pltpu.SemaphoreType.DMA,
        ],
        compiler_params=pltpu.CompilerParams(needs_layout_passes=False),
    )
    return f(seg, emb_table)


def _tc_body(seg_ref, ta_ref, tb_ref, out_ref):
    j = pl.program_id(0)
    c = jnp.sum(seg_ref[0, 0])
    tbl = jnp.concatenate([ta_ref[2:], tb_ref[:2]], axis=0)
    rows = jax.lax.broadcasted_iota(jnp.int32, (BS, 1), 0) + j * BS
    out_ref[0] = jnp.where(rows < c, tbl, 0.0)


def _tc_run(seg3, emb_table):
    return pl.pallas_call(
        _tc_body,
        grid=(S // BS, NTC),
        in_specs=[
            pl.BlockSpec((1, 1, S), lambda j, i: (i, 0, 0)),
            pl.BlockSpec((BS, D), lambda j, i: (j, 0)),
            pl.BlockSpec((BS, D), lambda j, i: (j + 1, 0)),
        ],
        out_specs=pl.BlockSpec((1, BS, D), lambda j, i: (i, j, 0)),
        out_shape=jax.ShapeDtypeStruct((NTC, S, D), jnp.float32),
        compiler_params=pltpu.CompilerParams(
            dimension_semantics=("arbitrary", "arbitrary")),
    )(seg3, emb_table, emb_table)


@jax.jit
def _run(seg, emb_table):
    tc_out = _tc_run(seg[:NTC].reshape(NTC, 1, S), emb_table)
    sc_out = _sc_run(seg, emb_table)
    return jnp.concatenate([tc_out, sc_out], axis=0)


def kernel(src, seg, emb_table):
    del src
    return _run(seg, emb_table)


# speedup vs baseline: 1.5383x; 1.5383x over previous
import functools

import jax
import jax.numpy as jnp
from jax import lax
from jax.experimental import pallas as pl
from jax.experimental.pallas import tpu as pltpu
from jax.experimental.pallas import tpu_sc as plsc

B, S, D = 8, 4096, 1024
NC, NS = 2, 16
NW = NC * NS
ROWS_PER_W = S // NW
CH = 32
NCHUNK = ROWS_PER_W // CH
ZH = 16
PH = 16
SSTRIP = 512
NSTRIP = S // SSTRIP
L = 16


def _body(seg_hbm, table_hbm, out_hbm,
          t0_v, t1_v, z_v, p0_v, p1_v, sg0_v, sg1_v,
          idx0_v, idx1_v, pidx0_v, pidx1_v,
          sem_l, sem_s, sem_w, sem_p, sem_b):
    wid = lax.axis_index("s") * NC + lax.axis_index("c")
    tbufs = [t0_v, t1_v]
    idxbufs = [idx0_v, idx1_v]
    sgbufs = [sg0_v, sg1_v]
    lane = lax.iota(jnp.int32, L)

    def chunk_j0(q):
        return (q * NW + wid) * CH

    def build_idx(ref, j0):
        ref[pl.ds(0, L)] = lane + (j0 + 2)
        ref[pl.ds(L, L)] = lane + (j0 + 2 + L)

    build_idx(idx0_v, chunk_j0(0))
    pltpu.async_copy(table_hbm.at[idx0_v], t0_v, sem_l)
    pltpu.async_copy(seg_hbm.at[:, pl.ds(0, SSTRIP)], sg0_v, sem_s)

    zvec = jnp.zeros((L,), jnp.float32)

    def zrow(r, _):
        def zcol(k, _):
            z_v[r, pl.ds(k * 4 * L, L)] = zvec
            z_v[r, pl.ds((k * 4 + 1) * L, L)] = zvec
            z_v[r, pl.ds((k * 4 + 2) * L, L)] = zvec
            z_v[r, pl.ds((k * 4 + 3) * L, L)] = zvec
            return 0
        return lax.fori_loop(0, D // L // 4, zcol, 0)

    lax.fori_loop(0, ZH, zrow, 0)

    accs = tuple(jnp.zeros((L,), jnp.int32) for _ in range(B))
    for k in range(NSTRIP):
        sg = sgbufs[k % 2]
        pltpu.make_async_copy(
            seg_hbm.at[:, pl.ds(k * SSTRIP, SSTRIP)], sg, sem_s).wait()
        if k + 1 < NSTRIP:
            pltpu.async_copy(
                seg_hbm.at[:, pl.ds((k + 1) * SSTRIP, SSTRIP)],
                sgbufs[(k + 1) % 2], sem_s)

        def red(t, acc):
            return tuple(acc[i] + sg[i, pl.ds(t * L, L)] for i in range(B))

        accs = lax.fori_loop(0, SSTRIP // L, red, accs)
    counts = [jnp.sum(a) for a in accs]

    def drain_bnd(n):
        def w(_, carry):
            pltpu.make_async_copy(p0_v, out_hbm.at[0, pl.ds(0, PH)],
                                  sem_b).wait()
            return carry
        lax.fori_loop(0, n, w, 0)

    bfired = jnp.int32(0)
    for i in range(B):
        c = counts[i]
        g = c // CH
        mine = jnp.logical_and(c % CH != 0, wid == i)

        @pl.when(mine)
        def _():
            j0b = g * CH
            jv0 = lane + j0b
            pidx0_v[pl.ds(0, L)] = jnp.where(jv0 < c, jv0 + 2, 0)
            jv1 = lane + (j0b + PH)
            pidx1_v[pl.ds(0, L)] = jnp.where(jv1 < c, jv1 + 2, 0)
            cp0 = pltpu.async_copy(table_hbm.at[pidx0_v], p0_v, sem_p)
            cp1 = pltpu.async_copy(table_hbm.at[pidx1_v], p1_v, sem_p)
            cp0.wait()
            cp1.wait()
            pltpu.async_copy(p0_v, out_hbm.at[i, pl.ds(j0b, PH)], sem_b)
            pltpu.async_copy(p1_v, out_hbm.at[i, pl.ds(j0b + PH, PH)],
                             sem_b)

        bfired = bfired + jnp.where(mine, 2, 0).astype(jnp.int32)

    def drain_writes(n):
        def w(_, carry):
            pltpu.make_async_copy(z_v, out_hbm.at[0, pl.ds(0, ZH)],
                                  sem_w).wait()
            return carry
        lax.fori_loop(0, n, w, 0)

    fired = jnp.int32(0)
    drained = jnp.int32(0)

    for q in range(NCHUNK):
        t_v = tbufs[q % 2]
        j0 = chunk_j0(q)
        pltpu.make_async_copy(table_hbm.at[idxbufs[q % 2]], t_v,
                              sem_l).wait()
        if q + 1 < NCHUNK:
            drain_writes(fired - drained)
            drained = fired
            build_idx(idxbufs[(q + 1) % 2], chunk_j0(q + 1))
            pltpu.async_copy(table_hbm.at[idxbufs[(q + 1) % 2]],
                             tbufs[(q + 1) % 2], sem_l)

        for i in range(B):
            c = counts[i]
            full = (j0 + CH) <= c
            empty = j0 >= c
            boundary = jnp.logical_and(jnp.logical_not(full),
                                       jnp.logical_not(empty))

            @pl.when(full)
            def _():
                pltpu.async_copy(t_v, out_hbm.at[i, pl.ds(j0, CH)], sem_w)

            @pl.when(empty)
            def _():
                pltpu.async_copy(z_v, out_hbm.at[i, pl.ds(j0, ZH)], sem_w)
                pltpu.async_copy(z_v, out_hbm.at[i, pl.ds(j0 + ZH, ZH)],
                                 sem_w)

            fired = fired + jnp.where(boundary, 0, 2).astype(jnp.int32)

    drain_writes(fired - drained)
    drain_bnd(bfired)


@functools.partial(jax.jit, static_argnames=())
def _run(seg, emb_table):
    mesh = plsc.VectorSubcoreMesh(core_axis_name="c", subcore_axis_name="s")
    f = pl.kernel(
        _body,
        out_type=jax.ShapeDtypeStruct((B, S, D), jnp.float32),
        mesh=mesh,
        scratch_types=[
            pltpu.VMEM((CH, D), jnp.float32),
            pltpu.VMEM((CH, D), jnp.float32),
            pltpu.VMEM((ZH, D), jnp.float32),
            pltpu.VMEM((PH, D), jnp.float32),
            pltpu.VMEM((PH, D), jnp.float32),
            pltpu.VMEM((B, SSTRIP), jnp.int32),
            pltpu.VMEM((B, SSTRIP), jnp.int32),
            pltpu.VMEM((CH,), jnp.int32),
            pltpu.VMEM((CH,), jnp.int32),
            pltpu.VMEM((PH,), jnp.int32),
            pltpu.VMEM((PH,), jnp.int32),
            pltpu.SemaphoreType.DMA,
            pltpu.SemaphoreType.DMA,
            pltpu.SemaphoreType.DMA,
            pltpu.SemaphoreType.DMA,
            pltpu.SemaphoreType.DMA,
        ],
        compiler_params=pltpu.CompilerParams(needs_layout_passes=False),
    )
    return f(seg, emb_table)


def kernel(src, seg, emb_table):
    del src
    return _run(seg, emb_table)


# speedup vs baseline: 1.7714x; 1.1515x over previous
import functools

import jax
import jax.numpy as jnp
from jax import lax
from jax.experimental import pallas as pl
from jax.experimental.pallas import tpu as pltpu
from jax.experimental.pallas import tpu_sc as plsc

B, S, D = 8, 4096, 1024
NC, NS = 2, 16
NW = NC * NS
ROWS_PER_W = S // NW
CH = 32
NCHUNK = ROWS_PER_W // CH
ZH = 16
PH = 16
SSTRIP = 512
NSTRIP = S // SSTRIP
L = 16


def _body(seg_hbm, table_hbm, out_hbm,
          t0_v, t1_v, z_v, p0_v, p1_v, sg0_v, sg1_v,
          idx0_v, idx1_v, pidx0_v, pidx1_v,
          sem_l, sem_s, sem_w, sem_p, sem_b):
    wid = lax.axis_index("s") * NC + lax.axis_index("c")
    tbufs = [t0_v, t1_v]
    idxbufs = [idx0_v, idx1_v]
    sgbufs = [sg0_v, sg1_v]
    lane = lax.iota(jnp.int32, L)

    def chunk_j0(q):
        return (q * NW + wid) * CH

    def build_idx(ref, j0):
        ref[pl.ds(0, L)] = lane + (j0 + 2)
        ref[pl.ds(L, L)] = lane + (j0 + 2 + L)

    build_idx(idx0_v, chunk_j0(0))
    pltpu.async_copy(table_hbm.at[idx0_v], t0_v, sem_l)
    pltpu.async_copy(seg_hbm.at[:, pl.ds(0, SSTRIP)], sg0_v, sem_s)

    zvec = jnp.zeros((L,), jnp.float32)

    def zrow(r, _):
        def zcol(k, _):
            z_v[r, pl.ds(k * 4 * L, L)] = zvec
            z_v[r, pl.ds((k * 4 + 1) * L, L)] = zvec
            z_v[r, pl.ds((k * 4 + 2) * L, L)] = zvec
            z_v[r, pl.ds((k * 4 + 3) * L, L)] = zvec
            return 0
        return lax.fori_loop(0, D // L // 4, zcol, 0)

    lax.fori_loop(0, ZH, zrow, 0)

    accs = tuple(jnp.zeros((L,), jnp.int32) for _ in range(B))
    for k in range(NSTRIP):
        sg = sgbufs[k % 2]
        pltpu.make_async_copy(
            seg_hbm.at[:, pl.ds(k * SSTRIP, SSTRIP)], sg, sem_s).wait()
        if k + 1 < NSTRIP:
            pltpu.async_copy(
                seg_hbm.at[:, pl.ds((k + 1) * SSTRIP, SSTRIP)],
                sgbufs[(k + 1) % 2], sem_s)

        def red(t, acc):
            return tuple(acc[i] + sg[i, pl.ds(t * L, L)] for i in range(B))

        accs = lax.fori_loop(0, SSTRIP // L, red, accs)
    counts = [jnp.sum(a) for a in accs]

    def drain_bnd(n):
        def w(_, carry):
            pltpu.make_async_copy(p0_v, out_hbm.at[0, pl.ds(0, PH)],
                                  sem_b).wait()
            return carry
        lax.fori_loop(0, n, w, 0)

    bfired = jnp.int32(0)
    for i in range(B):
        c = counts[i]
        g = c // CH
        mine = jnp.logical_and(c % CH != 0, wid == i)

        @pl.when(mine)
        def _():
            j0b = g * CH
            jv0 = lane + j0b
            pidx0_v[pl.ds(0, L)] = jnp.where(jv0 < c, jv0 + 2, 0)
            jv1 = lane + (j0b + PH)
            pidx1_v[pl.ds(0, L)] = jnp.where(jv1 < c, jv1 + 2, 0)
            cp0 = pltpu.async_copy(table_hbm.at[pidx0_v], p0_v, sem_p)
            cp1 = pltpu.async_copy(table_hbm.at[pidx1_v], p1_v, sem_p)
            cp0.wait()
            cp1.wait()
            pltpu.async_copy(p0_v, out_hbm.at[i, pl.ds(j0b, PH)], sem_b)
            pltpu.async_copy(p1_v, out_hbm.at[i, pl.ds(j0b + PH, PH)],
                             sem_b)

        bfired = bfired + jnp.where(mine, 2, 0).astype(jnp.int32)

    def drain_writes(n):
        def w(_, carry):
            pltpu.make_async_copy(z_v, out_hbm.at[0, pl.ds(0, ZH)],
                                  sem_w).wait()
            return carry
        lax.fori_loop(0, n, w, 0)

    fired = jnp.int32(0)
    drained = jnp.int32(0)

    def steal(g, carry):
        f, dr = carry
        j0 = g * CH
        for i in range(B):
            c = counts[i]
            m = (g * B + i) % 144
            zown = jnp.where(m < 64, 2 * (m % 16), 2 * ((m - 64) % 16) + 1)
            cond = jnp.logical_and(j0 >= c, zown == wid)

            @pl.when(cond)
            def _():
                pltpu.async_copy(z_v, out_hbm.at[i, pl.ds(j0, ZH)], sem_w)
                pltpu.async_copy(z_v, out_hbm.at[i, pl.ds(j0 + ZH, ZH)],
                                 sem_w)

            f = f + jnp.where(cond, 2, 0).astype(jnp.int32)
        over = jnp.maximum(f - dr - 40, 0)
        drain_writes(over)
        return (f, dr + over)

    fired, drained = lax.fori_loop(0, S // CH, steal, (fired, drained))

    for q in range(NCHUNK):
        t_v = tbufs[q % 2]
        j0 = chunk_j0(q)
        pltpu.make_async_copy(table_hbm.at[idxbufs[q % 2]], t_v,
                              sem_l).wait()
        if q + 1 < NCHUNK:
            drain_writes(fired - drained)
            drained = fired
            build_idx(idxbufs[(q + 1) % 2], chunk_j0(q + 1))
            pltpu.async_copy(table_hbm.at[idxbufs[(q + 1) % 2]],
                             tbufs[(q + 1) % 2], sem_l)

        for i in range(B):
            c = counts[i]
            full = (j0 + CH) <= c

            @pl.when(full)
            def _():
                pltpu.async_copy(t_v, out_hbm.at[i, pl.ds(j0, CH)], sem_w)

            fired = fired + jnp.where(full, 2, 0).astype(jnp.int32)

    drain_writes(fired - drained)
    drain_bnd(bfired)


@functools.partial(jax.jit, static_argnames=())
def _run(seg, emb_table):
    mesh = plsc.VectorSubcoreMesh(core_axis_name="c", subcore_axis_name="s")
    f = pl.kernel(
        _body,
        out_type=jax.ShapeDtypeStruct((B, S, D), jnp.float32),
        mesh=mesh,
        scratch_types=[
            pltpu.VMEM((CH, D), jnp.float32),
            pltpu.VMEM((CH, D), jnp.float32),
            pltpu.VMEM((ZH, D), jnp.float32),
            pltpu.VMEM((PH, D), jnp.float32),
            pltpu.VMEM((PH, D), jnp.float32),
            pltpu.VMEM((B, SSTRIP), jnp.int32),
            pltpu.VMEM((B, SSTRIP), jnp.int32),
            pltpu.VMEM((CH,), jnp.int32),
            pltpu.VMEM((CH,), jnp.int32),
            pltpu.VMEM((PH,), jnp.int32),
            pltpu.VMEM((PH,), jnp.int32),
            pltpu.SemaphoreType.DMA,
            pltpu.SemaphoreType.DMA,
            pltpu.SemaphoreType.DMA,
            pltpu.SemaphoreType.DMA,
            pltpu.SemaphoreType.DMA,
        ],
        compiler_params=pltpu.CompilerParams(needs_layout_passes=False),
    )
    return f(seg, emb_table)


def kernel(src, seg, emb_table):
    del src
    return _run(seg, emb_table)
